# baseline (speedup 1.0000x reference)
import jax
import jax.numpy as jnp
from jax.experimental import pallas as pl
from jax.experimental.pallas import tpu as pltpu

B, C, L = 16, 256, 1024
DIM_U, Q_EMB = 64, 64
INTER = DIM_U * Q_EMB
C2 = 2 * C
EPS = 1e-5
BB = 4
NCH = B // BB
DEPTH = 2


def _body(u_smem, x_any, e_any, w_any, t_any, vb_any, out_any,
          xbuf, ebuf, obuf, wbuf, tbuf, vbbuf,
          xsem, esem, osem, wsem, tsem, vbsem):
    def copy_in(i, slot):
        pltpu.make_async_copy(
            x_any.at[pl.ds(BB * i, BB)], xbuf.at[slot], xsem.at[slot]).start()
        pltpu.make_async_copy(
            e_any.at[pl.ds(BB * i, BB)], ebuf.at[slot], esem.at[slot]).start()

    pltpu.make_async_copy(w_any, wbuf, wsem).start()
    pltpu.make_async_copy(t_any, tbuf, tsem).start()
    pltpu.make_async_copy(vb_any, vbbuf, vbsem).start()
    copy_in(0, 0)
    pltpu.make_async_copy(w_any, wbuf, wsem).wait()
    pltpu.make_async_copy(t_any, tbuf, tsem).wait()
    pltpu.make_async_copy(vb_any, vbbuf, vbsem).wait()

    lane = jax.lax.broadcasted_iota(jnp.int32, (1, 128), 1)
    ones_row = jnp.ones((1, L), jnp.float32)
    for i in range(NCH):
        slot = i % DEPTH
        if i + 1 < NCH:
            copy_in(i + 1, (i + 1) % DEPTH)
        pltpu.make_async_copy(
            x_any.at[pl.ds(BB * i, BB)], xbuf.at[slot], xsem.at[slot]).wait()
        pltpu.make_async_copy(
            e_any.at[pl.ds(BB * i, BB)], ebuf.at[slot], esem.at[slot]).wait()
        if i >= DEPTH:
            pltpu.make_async_copy(
                obuf.at[slot], out_any.at[pl.ds(BB * (i - DEPTH), BB)],
                osem.at[slot]).wait()
        for j in range(BB):
            b = BB * i + j
            xb = xbuf[slot, j]
            mu = jnp.sum(xb, axis=1, keepdims=True) * (1.0 / L)
            var = jnp.sum(xb * xb, axis=1, keepdims=True) * (1.0 / L) - mu * mu
            rstd = jax.lax.rsqrt(var + EPS)
            acc = jnp.zeros((C2, 128), jnp.float32)
            for k in range(DIM_U // 2):
                s0 = u_smem[b, 2 * k]
                s1 = u_smem[b, 2 * k + 1]
                m = jnp.where(lane < Q_EMB, s0, s1)
                acc = acc + wbuf[:, 128 * k:128 * (k + 1)] * m
            lhs = jnp.concatenate([acc, vbbuf[...]], axis=1)
            rhs = jnp.concatenate(
                [ebuf[slot, j], ebuf[slot, j], tbuf[b], ones_row], axis=0)
            params = jnp.dot(lhs, rhs, preferred_element_type=jnp.float32)
            nx = (xb - mu) * rstd
            obuf[slot, j] = params[:C] * nx + params[C:]
        pltpu.make_async_copy(
            obuf.at[slot], out_any.at[pl.ds(BB * i, BB)], osem.at[slot]).start()
    for i in range(NCH - DEPTH, NCH):
        slot = i % DEPTH
        pltpu.make_async_copy(
            obuf.at[slot], out_any.at[pl.ds(BB * i, BB)], osem.at[slot]).wait()


def kernel(x, u_i, e_qid, t, W, V, bias):
    vb = jnp.concatenate(
        [V, bias[:, None] + (jnp.arange(C2) < C).astype(jnp.float32)[:, None]],
        axis=1)
    return pl.pallas_call(
        _body,
        out_shape=jax.ShapeDtypeStruct((B, C, L), jnp.float32),
        in_specs=[
            pl.BlockSpec(memory_space=pltpu.SMEM),
            pl.BlockSpec(memory_space=pl.ANY),
            pl.BlockSpec(memory_space=pl.ANY),
            pl.BlockSpec(memory_space=pl.ANY),
            pl.BlockSpec(memory_space=pl.ANY),
            pl.BlockSpec(memory_space=pl.ANY),
        ],
        out_specs=pl.BlockSpec(memory_space=pl.ANY),
        scratch_shapes=[
            pltpu.VMEM((DEPTH, BB, C, L), jnp.float32),
            pltpu.VMEM((DEPTH, BB, Q_EMB, L), jnp.float32),
            pltpu.VMEM((DEPTH, BB, C, L), jnp.float32),
            pltpu.VMEM((C2, INTER), jnp.float32),
            pltpu.VMEM((B, 1, L), jnp.float32),
            pltpu.VMEM((C2, 2), jnp.float32),
            pltpu.SemaphoreType.DMA((DEPTH,)),
            pltpu.SemaphoreType.DMA((DEPTH,)),
            pltpu.SemaphoreType.DMA((DEPTH,)),
            pltpu.SemaphoreType.DMA,
            pltpu.SemaphoreType.DMA,
            pltpu.SemaphoreType.DMA,
        ],
        name="adain_fused",
    )(u_i, x, e_qid, W, t, vb)


# speedup vs baseline: 1.0220x; 1.0220x over previous
import jax
import jax.numpy as jnp
from jax.experimental import pallas as pl
from jax.experimental.pallas import tpu as pltpu

B, C, L = 16, 256, 1024
DIM_U, Q_EMB = 64, 64
INTER = DIM_U * Q_EMB
C2 = 2 * C
EPS = 1e-5
BB = 2
NCH = B // BB
DEPTH = 4


def _body(u_smem, x_any, e_any, w_any, t_any, vb_any, out_any,
          xbuf, ebuf, obuf, wbuf, tbuf, vbbuf,
          xsem, esem, osem, wsem, tsem, vbsem):
    def copy_in(i, slot):
        pltpu.make_async_copy(
            x_any.at[pl.ds(BB * i, BB)], xbuf.at[slot], xsem.at[slot]).start()
        pltpu.make_async_copy(
            e_any.at[pl.ds(BB * i, BB)], ebuf.at[slot], esem.at[slot]).start()

    pltpu.make_async_copy(w_any, wbuf, wsem).start()
    pltpu.make_async_copy(t_any, tbuf, tsem).start()
    pltpu.make_async_copy(vb_any, vbbuf, vbsem).start()
    for p in range(DEPTH - 1):
        copy_in(p, p)
    pltpu.make_async_copy(w_any, wbuf, wsem).wait()
    pltpu.make_async_copy(t_any, tbuf, tsem).wait()
    pltpu.make_async_copy(vb_any, vbbuf, vbsem).wait()

    lane = jax.lax.broadcasted_iota(jnp.int32, (1, 128), 1)
    ones_row = jnp.ones((1, L), jnp.float32)
    for i in range(NCH):
        slot = i % DEPTH
        if i + DEPTH - 1 < NCH:
            copy_in(i + DEPTH - 1, (i + DEPTH - 1) % DEPTH)
        pltpu.make_async_copy(
            x_any.at[pl.ds(BB * i, BB)], xbuf.at[slot], xsem.at[slot]).wait()
        pltpu.make_async_copy(
            e_any.at[pl.ds(BB * i, BB)], ebuf.at[slot], esem.at[slot]).wait()
        if i >= DEPTH:
            pltpu.make_async_copy(
                obuf.at[slot], out_any.at[pl.ds(BB * (i - DEPTH), BB)],
                osem.at[slot]).wait()
        for j in range(BB):
            b = BB * i + j
            xb = xbuf[slot, j]
            mu = jnp.sum(xb, axis=1, keepdims=True) * (1.0 / L)
            var = jnp.sum(xb * xb, axis=1, keepdims=True) * (1.0 / L) - mu * mu
            rstd = jax.lax.rsqrt(var + EPS)
            acc = jnp.zeros((C2, 128), jnp.float32)
            for k in range(DIM_U // 2):
                s0 = u_smem[b, 2 * k]
                s1 = u_smem[b, 2 * k + 1]
                m = jnp.where(lane < Q_EMB, s0, s1)
                acc = acc + wbuf[:, 128 * k:128 * (k + 1)] * m
            lhs = jnp.concatenate([acc, vbbuf[...]], axis=1)
            rhs = jnp.concatenate(
                [ebuf[slot, j], ebuf[slot, j], tbuf[b], ones_row], axis=0)
            params = jnp.dot(lhs, rhs, preferred_element_type=jnp.float32)
            nx = (xb - mu) * rstd
            obuf[slot, j] = params[:C] * nx + params[C:]
        pltpu.make_async_copy(
            obuf.at[slot], out_any.at[pl.ds(BB * i, BB)], osem.at[slot]).start()
    for i in range(NCH - DEPTH, NCH):
        slot = i % DEPTH
        pltpu.make_async_copy(
            obuf.at[slot], out_any.at[pl.ds(BB * i, BB)], osem.at[slot]).wait()


def kernel(x, u_i, e_qid, t, W, V, bias):
    vb = jnp.concatenate(
        [V, bias[:, None] + (jnp.arange(C2) < C).astype(jnp.float32)[:, None]],
        axis=1)
    return pl.pallas_call(
        _body,
        out_shape=jax.ShapeDtypeStruct((B, C, L), jnp.float32),
        in_specs=[
            pl.BlockSpec(memory_space=pltpu.SMEM),
            pl.BlockSpec(memory_space=pl.ANY),
            pl.BlockSpec(memory_space=pl.ANY),
            pl.BlockSpec(memory_space=pl.ANY),
            pl.BlockSpec(memory_space=pl.ANY),
            pl.BlockSpec(memory_space=pl.ANY),
        ],
        out_specs=pl.BlockSpec(memory_space=pl.ANY),
        scratch_shapes=[
            pltpu.VMEM((DEPTH, BB, C, L), jnp.float32),
            pltpu.VMEM((DEPTH, BB, Q_EMB, L), jnp.float32),
            pltpu.VMEM((DEPTH, BB, C, L), jnp.float32),
            pltpu.VMEM((C2, INTER), jnp.float32),
            pltpu.VMEM((B, 1, L), jnp.float32),
            pltpu.VMEM((C2, 2), jnp.float32),
            pltpu.SemaphoreType.DMA((DEPTH,)),
            pltpu.SemaphoreType.DMA((DEPTH,)),
            pltpu.SemaphoreType.DMA((DEPTH,)),
            pltpu.SemaphoreType.DMA,
            pltpu.SemaphoreType.DMA,
            pltpu.SemaphoreType.DMA,
        ],
        name="adain_fused",
    )(u_i, x, e_qid, W, t, vb)


# speedup vs baseline: 1.0528x; 1.0301x over previous
import jax
import jax.numpy as jnp
from jax.experimental import pallas as pl
from jax.experimental.pallas import tpu as pltpu

B, C, L = 16, 256, 1024
DIM_U, Q_EMB = 64, 64
INTER = DIM_U * Q_EMB
C2 = 2 * C
EPS = 1e-5
BB = 2
NCH = B // BB
DEPTH = 3


def _body(u_smem, x_any, e_any, w_any, t_any, vb_any, out_any,
          xbuf, ebuf, obuf, wbuf, tbuf, vbbuf,
          xsem, esem, osem, wsem, tsem, vbsem):
    def copy_in(i, slot):
        pltpu.make_async_copy(
            x_any.at[pl.ds(BB * i, BB)], xbuf.at[slot], xsem.at[slot]).start()
        pltpu.make_async_copy(
            e_any.at[pl.ds(BB * i, BB)], ebuf.at[slot], esem.at[slot]).start()

    pltpu.make_async_copy(w_any, wbuf, wsem).start()
    pltpu.make_async_copy(t_any, tbuf, tsem).start()
    pltpu.make_async_copy(vb_any, vbbuf, vbsem).start()
    for p in range(DEPTH - 1):
        copy_in(p, p)
    pltpu.make_async_copy(w_any, wbuf, wsem).wait()
    pltpu.make_async_copy(t_any, tbuf, tsem).wait()
    pltpu.make_async_copy(vb_any, vbbuf, vbsem).wait()

    lane = jax.lax.broadcasted_iota(jnp.int32, (1, 128), 1)
    ones_row = jnp.ones((1, L), jnp.float32)
    for i in range(NCH):
        slot = i % DEPTH
        if i + DEPTH - 1 < NCH:
            copy_in(i + DEPTH - 1, (i + DEPTH - 1) % DEPTH)
        pltpu.make_async_copy(
            x_any.at[pl.ds(BB * i, BB)], xbuf.at[slot], xsem.at[slot]).wait()
        pltpu.make_async_copy(
            e_any.at[pl.ds(BB * i, BB)], ebuf.at[slot], esem.at[slot]).wait()
        if i >= DEPTH:
            pltpu.make_async_copy(
                obuf.at[slot], out_any.at[pl.ds(BB * (i - DEPTH), BB)],
                osem.at[slot]).wait()
        for j in range(BB):
            b = BB * i + j
            xb = xbuf[slot, j]
            mu = jnp.sum(xb, axis=1, keepdims=True) * (1.0 / L)
            var = jnp.sum(xb * xb, axis=1, keepdims=True) * (1.0 / L) - mu * mu
            rstd = jax.lax.rsqrt(var + EPS)
            acc = jnp.zeros((C2, 128), jnp.float32)
            for k in range(DIM_U // 2):
                s0 = u_smem[b, 2 * k]
                s1 = u_smem[b, 2 * k + 1]
                m = jnp.where(lane < Q_EMB, s0, s1)
                acc = acc + wbuf[:, 128 * k:128 * (k + 1)] * m
            lhs = jnp.concatenate([acc, vbbuf[...]], axis=1)
            rhs = jnp.concatenate(
                [ebuf[slot, j], ebuf[slot, j], tbuf[b], ones_row], axis=0)
            params = jnp.dot(lhs, rhs, preferred_element_type=jnp.float32)
            nx = (xb - mu) * rstd
            obuf[slot, j] = params[:C] * nx + params[C:]
        pltpu.make_async_copy(
            obuf.at[slot], out_any.at[pl.ds(BB * i, BB)], osem.at[slot]).start()
    for i in range(NCH - DEPTH, NCH):
        slot = i % DEPTH
        pltpu.make_async_copy(
            obuf.at[slot], out_any.at[pl.ds(BB * i, BB)], osem.at[slot]).wait()


def kernel(x, u_i, e_qid, t, W, V, bias):
    vb = jnp.concatenate(
        [V, bias[:, None] + (jnp.arange(C2) < C).astype(jnp.float32)[:, None]],
        axis=1)
    return pl.pallas_call(
        _body,
        out_shape=jax.ShapeDtypeStruct((B, C, L), jnp.float32),
        in_specs=[
            pl.BlockSpec(memory_space=pltpu.SMEM),
            pl.BlockSpec(memory_space=pl.ANY),
            pl.BlockSpec(memory_space=pl.ANY),
            pl.BlockSpec(memory_space=pl.ANY),
            pl.BlockSpec(memory_space=pl.ANY),
            pl.BlockSpec(memory_space=pl.ANY),
        ],
        out_specs=pl.BlockSpec(memory_space=pl.ANY),
        scratch_shapes=[
            pltpu.VMEM((DEPTH, BB, C, L), jnp.float32),
            pltpu.VMEM((DEPTH, BB, Q_EMB, L), jnp.float32),
            pltpu.VMEM((DEPTH, BB, C, L), jnp.float32),
            pltpu.VMEM((C2, INTER), jnp.float32),
            pltpu.VMEM((B, 1, L), jnp.float32),
            pltpu.VMEM((C2, 2), jnp.float32),
            pltpu.SemaphoreType.DMA((DEPTH,)),
            pltpu.SemaphoreType.DMA((DEPTH,)),
            pltpu.SemaphoreType.DMA((DEPTH,)),
            pltpu.SemaphoreType.DMA,
            pltpu.SemaphoreType.DMA,
            pltpu.SemaphoreType.DMA,
        ],
        name="adain_fused",
    )(u_i, x, e_qid, W, t, vb)
